# baseline (speedup 1.0000x reference)
import jax
import jax.numpy as jnp
from jax import lax
from jax.experimental import pallas as pl
from jax.experimental.pallas import tpu as pltpu
from jax.experimental.pallas import tpu_sc as plsc

CTX = 2048
DIM = 64
BATCH = 4

NC = 2
NS = 16
NW = NC * NS
COLS = CTX // NW
LANES = 16


def _emb_body(x_hbm, tok_hbm, pos_hbm, out_hbm, idx_v, rows_v, pos_v, sem):
    c = lax.axis_index("c")
    s = lax.axis_index("s")
    wid = s * NC + c
    col0 = wid * COLS

    pltpu.sync_copy(pos_hbm.at[pl.ds(col0, COLS)], pos_v)
    for b in range(BATCH):
        pltpu.sync_copy(x_hbm.at[pl.ds(b * CTX + col0, COLS)], idx_v.at[b])

    copies = [
        pltpu.async_copy(tok_hbm.at[idx_v.at[b]], rows_v.at[b], sem)
        for b in range(BATCH)
    ]
    for cp in copies:
        cp.wait()

    def body(r, carry):
        for j in range(DIM // LANES):
            chunk = pos_v[r, pl.ds(j * LANES, LANES)]
            for b in range(BATCH):
                plsc.addupdate(rows_v.at[b, r, pl.ds(j * LANES, LANES)], chunk)
        return carry

    lax.fori_loop(0, COLS, body, 0)

    for b in range(BATCH):
        pltpu.sync_copy(rows_v.at[b], out_hbm.at[pl.ds(b * CTX + col0, COLS)])


@jax.jit
def kernel(x, token_table, pos_table):
    xf = x.reshape(-1).astype(jnp.int32)
    mesh = plsc.VectorSubcoreMesh(core_axis_name="c", subcore_axis_name="s")
    out = pl.kernel(
        _emb_body,
        out_type=jax.ShapeDtypeStruct((BATCH * CTX, DIM), jnp.float32),
        mesh=mesh,
        scratch_types=[
            pltpu.VMEM((BATCH, COLS), jnp.int32),
            pltpu.VMEM((BATCH, COLS, DIM), jnp.float32),
            pltpu.VMEM((COLS, DIM), jnp.float32),
            pltpu.SemaphoreType.DMA,
        ],
        compiler_params=pltpu.CompilerParams(use_tc_tiling_on_sc=False),
    )(xf, token_table, pos_table)
    return out.reshape(BATCH, CTX, DIM)


# speedup vs baseline: 5.2561x; 5.2561x over previous
import jax
import jax.numpy as jnp
from jax import lax
from jax.experimental import pallas as pl
from jax.experimental.pallas import tpu as pltpu
from jax.experimental.pallas import tpu_sc as plsc

CTX = 2048
DIM = 64
BATCH = 4

NC = 2
NS = 16
NW = NC * NS
CBLK = 128
NB_PER_W = 2
TOK = NB_PER_W * CBLK
LANES = 16
NBUF = 8
NGRP = TOK // NBUF


def _emb_body(x_hbm, tokT_hbm, posT_hbm, out_hbm, idx_v, tcol_v, outT_v,
              idx_s, *sems):
    c = lax.axis_index("c")
    s = lax.axis_index("s")
    wid = s * NC + c
    cb = wid % (CTX // CBLK)
    bb = wid // (CTX // CBLK)
    c0 = cb * CBLK

    for i in range(NB_PER_W):
        pltpu.sync_copy(
            x_hbm.at[pl.ds(NB_PER_W * bb + i, 1), pl.ds(c0, CBLK)],
            idx_v.at[pl.ds(i, 1)],
        )

    for i in range(NB_PER_W):
        pltpu.sync_copy(
            posT_hbm.at[:, pl.ds(c0, CBLK)],
            outT_v.at[:, pl.ds(i * CBLK, CBLK)],
        )

    for k in range(TOK // LANES):
        vec = idx_v[k // (CBLK // LANES), pl.ds((k % (CBLK // LANES)) * LANES, LANES)]
        for i in range(LANES):
            idx_s[k * LANES + i] = vec[i]

    iotas = [lax.iota(jnp.int32, LANES) + kk * LANES for kk in range(DIM // LANES)]

    def fire(t, b):
        v = idx_s[t]
        off = pl.multiple_of((v >> 7) * 128, 128)
        return pltpu.async_copy(
            tokT_hbm.at[:, pl.ds(off, 128)], tcol_v.at[b], sems[b]
        )

    def process(t, b):
        v = idx_s[t]
        vl = jnp.full((LANES,), v & 127, jnp.int32)
        bsp = jnp.full((LANES,), b, jnp.int32)
        tsp = jnp.full((LANES,), t, jnp.int32)
        for kk in range(DIM // LANES):
            col = plsc.load_gather(tcol_v, [bsp, iotas[kk], vl])
            plsc.addupdate_scatter(outT_v, [iotas[kk], tsp], col)

    def drain(b):
        pltpu.make_async_copy(
            tokT_hbm.at[:, pl.ds(0, 128)], tcol_v.at[b], sems[b]
        ).wait()

    for b in range(NBUF):
        fire(b, b)

    def group(g, carry):
        for b in range(NBUF):
            t = g * NBUF + b
            drain(b)
            process(t, b)
            fire(t + NBUF, b)
        return carry

    lax.fori_loop(0, NGRP - 1, group, 0)

    for b in range(NBUF):
        t = (NGRP - 1) * NBUF + b
        drain(b)
        process(t, b)

    for i in range(NB_PER_W):
        pltpu.sync_copy(
            outT_v.at[:, pl.ds(i * CBLK, CBLK)],
            out_hbm.at[NB_PER_W * bb + i, :, pl.ds(c0, CBLK)],
        )


@jax.jit
def kernel(x, token_table, pos_table):
    tokT = token_table.T
    posT = pos_table.T
    mesh = plsc.VectorSubcoreMesh(core_axis_name="c", subcore_axis_name="s")
    out = pl.kernel(
        _emb_body,
        out_type=jax.ShapeDtypeStruct((BATCH, DIM, CTX), jnp.float32),
        mesh=mesh,
        scratch_types=[
            pltpu.VMEM((NB_PER_W, CBLK), jnp.int32),
            pltpu.VMEM((NBUF, DIM, 128), jnp.float32),
            pltpu.VMEM((DIM, TOK), jnp.float32),
            pltpu.SMEM((TOK,), jnp.int32),
        ] + [pltpu.SemaphoreType.DMA] * NBUF,
        compiler_params=pltpu.CompilerParams(needs_layout_passes=False),
    )(x, tokT, posT)
    return jnp.transpose(out, (0, 2, 1))
